# baseline (speedup 1.0000x reference)
import jax
import jax.numpy as jnp
from jax.experimental import pallas as pl
from jax.experimental.pallas import tpu as pltpu

_NB = 128
_BW = 16
_E = 4096

_EXP_MASK = -65536


def _kernel(x_ref, bpt_ref, v17t_ref, o_ref, bnd_s, tab_s):
    f32 = jnp.float32
    bf16 = jnp.bfloat16

    @pl.when(pl.program_id(0) == 0)
    def _prep():
        def split3(a):
            bits = jax.lax.bitcast_convert_type(a, jnp.int32)
            hi = jax.lax.bitcast_convert_type(bits & _EXP_MASK, f32)
            r1 = a - hi
            b1 = jax.lax.bitcast_convert_type(r1, jnp.int32)
            mid = jax.lax.bitcast_convert_type(b1 & _EXP_MASK, f32)
            lo = r1 - mid
            return hi, mid, lo

        bh, bm, bl = split3(bpt_ref[...])
        vh, vm, vl = split3(v17t_ref[...])
        z7 = jnp.zeros((7, _NB), f32)
        table_t = jnp.concatenate([bh, bm, bl, vh, z7, vm, z7, vl], axis=0)
        tab_s[...] = table_t.astype(bf16)

        bnd_col = jnp.transpose(bpt_ref[_BW - 1 : _BW, :])
        bnd_s[...] = jnp.broadcast_to(bnd_col, (_NB, _E))

    xrow = x_ref[0]
    h1 = (xrow > bnd_s[...]).astype(bf16)
    h1p = jnp.concatenate([jnp.ones((1, _E), bf16), h1[: _NB - 1]], axis=0)
    onehot = h1p - h1

    g = jnp.dot(tab_s[...], onehot, preferred_element_type=f32)
    bp_row = (g[0:16] + g[16:32]) + g[32:48]
    v_row = (g[48:65] + g[72:89]) + g[96:113]

    cmp = (xrow > bp_row).astype(f32)
    dv = v_row[1:17] - v_row[0:16]
    sel = v_row[0:1] + jnp.sum(cmp * dv, axis=0, keepdims=True)

    bp_last = bpt_ref[_BW - 1, _NB - 1]
    v_last = v17t_ref[_BW, _NB - 1]
    out = sel + (xrow > bp_last).astype(f32) * v_last
    o_ref[...] = out.reshape(1, 1, _E)


def kernel(x, breakpoints, values):
    B = x.shape[0]
    n = breakpoints.shape[0]
    steps = B // _E

    bp_r = breakpoints.reshape(_NB, _BW)
    bp_t = bp_r.T
    v_main = values[:n, 0].reshape(_NB, _BW)
    v_ext = values[1 : n + 1, 0].reshape(_NB, _BW)
    v17_t = jnp.concatenate([v_main, v_ext[:, _BW - 1 :]], axis=1).T

    x3 = x.reshape(steps, 1, _E)

    out = pl.pallas_call(
        _kernel,
        out_shape=jax.ShapeDtypeStruct((steps, 1, _E), jnp.float32),
        grid=(steps,),
        in_specs=[
            pl.BlockSpec((1, 1, _E), lambda i: (i, 0, 0)),
            pl.BlockSpec((_BW, _NB), lambda i: (0, 0)),
            pl.BlockSpec((_BW + 1, _NB), lambda i: (0, 0)),
        ],
        out_specs=pl.BlockSpec((1, 1, _E), lambda i: (i, 0, 0)),
        scratch_shapes=[
            pltpu.VMEM((_NB, _E), jnp.float32),
            pltpu.VMEM((113, _NB), jnp.bfloat16),
        ],
        compiler_params=pltpu.CompilerParams(
            dimension_semantics=("arbitrary",),
        ),
        name="stepnet_lookup",
    )(x3, bp_t, v17_t)
    return out.reshape(B, 1)


# speedup vs baseline: 1.0739x; 1.0739x over previous
import jax
import jax.numpy as jnp
from jax.experimental import pallas as pl
from jax.experimental.pallas import tpu as pltpu

_NB = 128
_BW = 16
_E = 8192

_EXP_MASK = -65536


def _kernel(x_ref, bpt_ref, v17t_ref, o_ref, bnd_s, tab_s):
    f32 = jnp.float32
    bf16 = jnp.bfloat16

    @pl.when(pl.program_id(0) == 0)
    def _prep():
        def split3(a):
            bits = jax.lax.bitcast_convert_type(a, jnp.int32)
            hi = jax.lax.bitcast_convert_type(bits & _EXP_MASK, f32)
            r1 = a - hi
            b1 = jax.lax.bitcast_convert_type(r1, jnp.int32)
            mid = jax.lax.bitcast_convert_type(b1 & _EXP_MASK, f32)
            lo = r1 - mid
            return hi, mid, lo

        bh, bm, bl = split3(bpt_ref[...])
        vh, vm, vl = split3(v17t_ref[...])
        z7 = jnp.zeros((7, _NB), f32)
        table_t = jnp.concatenate([bh, bm, bl, vh, z7, vm, z7, vl], axis=0)
        tab_s[...] = table_t.astype(bf16)

        bnd_col = jnp.transpose(bpt_ref[_BW - 1 : _BW, :])
        bnd_s[...] = jnp.broadcast_to(bnd_col, (_NB, _E))

    xrow = x_ref[0]
    h1 = (xrow > bnd_s[...]).astype(bf16)
    h1p = jnp.concatenate([jnp.ones((1, _E), bf16), h1[: _NB - 1]], axis=0)
    onehot = h1p - h1

    g = jnp.dot(tab_s[...], onehot, preferred_element_type=f32)
    bp_row = (g[0:16] + g[16:32]) + g[32:48]
    v_row = (g[48:65] + g[72:89]) + g[96:113]

    cmp = (xrow > bp_row).astype(f32)
    dv = v_row[1:17] - v_row[0:16]
    sel = v_row[0:1] + jnp.sum(cmp * dv, axis=0, keepdims=True)

    bp_last = bpt_ref[_BW - 1, _NB - 1]
    v_last = v17t_ref[_BW, _NB - 1]
    out = sel + (xrow > bp_last).astype(f32) * v_last
    o_ref[...] = out.reshape(1, 1, _E)


def kernel(x, breakpoints, values):
    B = x.shape[0]
    n = breakpoints.shape[0]
    steps = B // _E

    bp_r = breakpoints.reshape(_NB, _BW)
    bp_t = bp_r.T
    v_main = values[:n, 0].reshape(_NB, _BW)
    v_ext = values[1 : n + 1, 0].reshape(_NB, _BW)
    v17_t = jnp.concatenate([v_main, v_ext[:, _BW - 1 :]], axis=1).T

    x3 = x.reshape(steps, 1, _E)

    out = pl.pallas_call(
        _kernel,
        out_shape=jax.ShapeDtypeStruct((steps, 1, _E), jnp.float32),
        grid=(steps,),
        in_specs=[
            pl.BlockSpec((1, 1, _E), lambda i: (i, 0, 0)),
            pl.BlockSpec((_BW, _NB), lambda i: (0, 0)),
            pl.BlockSpec((_BW + 1, _NB), lambda i: (0, 0)),
        ],
        out_specs=pl.BlockSpec((1, 1, _E), lambda i: (i, 0, 0)),
        scratch_shapes=[
            pltpu.VMEM((_NB, _E), jnp.float32),
            pltpu.VMEM((113, _NB), jnp.bfloat16),
        ],
        compiler_params=pltpu.CompilerParams(
            dimension_semantics=("arbitrary",),
        ),
        name="stepnet_lookup",
    )(x3, bp_t, v17_t)
    return out.reshape(B, 1)


# speedup vs baseline: 1.0786x; 1.0044x over previous
import jax
import jax.numpy as jnp
from jax.experimental import pallas as pl
from jax.experimental.pallas import tpu as pltpu

_NB = 128
_BW = 16
_E = 16384

_EXP_MASK = -65536


def _kernel(x_ref, bpt_ref, v17t_ref, o_ref, bnd_s, tab_s):
    f32 = jnp.float32
    bf16 = jnp.bfloat16

    @pl.when(pl.program_id(0) == 0)
    def _prep():
        def split3(a):
            bits = jax.lax.bitcast_convert_type(a, jnp.int32)
            hi = jax.lax.bitcast_convert_type(bits & _EXP_MASK, f32)
            r1 = a - hi
            b1 = jax.lax.bitcast_convert_type(r1, jnp.int32)
            mid = jax.lax.bitcast_convert_type(b1 & _EXP_MASK, f32)
            lo = r1 - mid
            return hi, mid, lo

        bh, bm, bl = split3(bpt_ref[...])
        vh, vm, vl = split3(v17t_ref[...])
        z7 = jnp.zeros((7, _NB), f32)
        table_t = jnp.concatenate([bh, bm, bl, vh, z7, vm, z7, vl], axis=0)
        tab_s[...] = table_t.astype(bf16)

        bnd_col = jnp.transpose(bpt_ref[_BW - 1 : _BW, :])
        bnd_s[...] = jnp.broadcast_to(bnd_col, (_NB, _E))

    xrow = x_ref[0]
    h1 = (xrow > bnd_s[...]).astype(bf16)
    h1p = jnp.concatenate([jnp.ones((1, _E), bf16), h1[: _NB - 1]], axis=0)
    onehot = h1p - h1

    g = jnp.dot(tab_s[...], onehot, preferred_element_type=f32)
    bp_row = (g[0:16] + g[16:32]) + g[32:48]
    v_row = (g[48:65] + g[72:89]) + g[96:113]

    cmp = (xrow > bp_row).astype(f32)
    dv = v_row[1:17] - v_row[0:16]
    sel = v_row[0:1] + jnp.sum(cmp * dv, axis=0, keepdims=True)

    bp_last = bpt_ref[_BW - 1, _NB - 1]
    v_last = v17t_ref[_BW, _NB - 1]
    out = sel + (xrow > bp_last).astype(f32) * v_last
    o_ref[...] = out.reshape(1, 1, _E)


def kernel(x, breakpoints, values):
    B = x.shape[0]
    n = breakpoints.shape[0]
    steps = B // _E

    bp_r = breakpoints.reshape(_NB, _BW)
    bp_t = bp_r.T
    v_main = values[:n, 0].reshape(_NB, _BW)
    v_ext = values[1 : n + 1, 0].reshape(_NB, _BW)
    v17_t = jnp.concatenate([v_main, v_ext[:, _BW - 1 :]], axis=1).T

    x3 = x.reshape(steps, 1, _E)

    out = pl.pallas_call(
        _kernel,
        out_shape=jax.ShapeDtypeStruct((steps, 1, _E), jnp.float32),
        grid=(steps,),
        in_specs=[
            pl.BlockSpec((1, 1, _E), lambda i: (i, 0, 0)),
            pl.BlockSpec((_BW, _NB), lambda i: (0, 0)),
            pl.BlockSpec((_BW + 1, _NB), lambda i: (0, 0)),
        ],
        out_specs=pl.BlockSpec((1, 1, _E), lambda i: (i, 0, 0)),
        scratch_shapes=[
            pltpu.VMEM((_NB, _E), jnp.float32),
            pltpu.VMEM((113, _NB), jnp.bfloat16),
        ],
        compiler_params=pltpu.CompilerParams(
            dimension_semantics=("arbitrary",),
        ),
        name="stepnet_lookup",
    )(x3, bp_t, v17_t)
    return out.reshape(B, 1)


# speedup vs baseline: 1.2720x; 1.1793x over previous
import jax
import jax.numpy as jnp
from jax.experimental import pallas as pl
from jax.experimental.pallas import tpu as pltpu

_NB = 64
_BW = 32
_E = 16384

_VROWS = _BW + 1
_VPAD = (-_VROWS) % 8
_VSTRIDE = _VROWS + _VPAD
_V0 = 3 * _BW
_TROWS = _V0 + 2 * _VSTRIDE + _VROWS

_EXP_MASK = -65536


def _kernel(x_ref, bpt_ref, vt_ref, o_ref, bnd_s, tab_s):
    f32 = jnp.float32
    bf16 = jnp.bfloat16

    @pl.when(pl.program_id(0) == 0)
    def _prep():
        def split3(a):
            bits = jax.lax.bitcast_convert_type(a, jnp.int32)
            hi = jax.lax.bitcast_convert_type(bits & _EXP_MASK, f32)
            r1 = a - hi
            b1 = jax.lax.bitcast_convert_type(r1, jnp.int32)
            mid = jax.lax.bitcast_convert_type(b1 & _EXP_MASK, f32)
            lo = r1 - mid
            return hi, mid, lo

        bh, bm, bl = split3(bpt_ref[...])
        vh, vm, vl = split3(vt_ref[...])
        zp = jnp.zeros((_VPAD, _NB), f32)
        table_t = jnp.concatenate([bh, bm, bl, vh, zp, vm, zp, vl], axis=0)
        tab_s[...] = table_t.astype(bf16)

        bnd_col = jnp.transpose(bpt_ref[_BW - 1 : _BW, :])
        bnd_s[...] = jnp.broadcast_to(bnd_col, (_NB, _E))

    xrow = x_ref[0]
    h1 = (xrow > bnd_s[...]).astype(bf16)
    h1p = jnp.concatenate([jnp.ones((1, _E), bf16), h1[: _NB - 1]], axis=0)
    onehot = h1p - h1

    g = jnp.dot(tab_s[...], onehot, preferred_element_type=f32)
    bp_row = (g[0:_BW] + g[_BW : 2 * _BW]) + g[2 * _BW : 3 * _BW]
    v_row = (
        g[_V0 : _V0 + _VROWS]
        + g[_V0 + _VSTRIDE : _V0 + _VSTRIDE + _VROWS]
        + g[_V0 + 2 * _VSTRIDE : _V0 + 2 * _VSTRIDE + _VROWS]
    )

    cmp = (xrow > bp_row).astype(f32)
    dv = v_row[1:_VROWS] - v_row[: _BW]
    sel = v_row[0:1] + jnp.sum(cmp * dv, axis=0, keepdims=True)

    bp_last = bpt_ref[_BW - 1, _NB - 1]
    v_last = vt_ref[_BW, _NB - 1]
    out = sel + (xrow > bp_last).astype(f32) * v_last
    o_ref[...] = out.reshape(1, 1, _E)


def kernel(x, breakpoints, values):
    B = x.shape[0]
    n = breakpoints.shape[0]
    steps = B // _E

    bp_r = breakpoints.reshape(_NB, _BW)
    bp_t = bp_r.T
    v_main = values[:n, 0].reshape(_NB, _BW)
    v_ext = values[1 : n + 1, 0].reshape(_NB, _BW)
    v_t = jnp.concatenate([v_main, v_ext[:, _BW - 1 :]], axis=1).T

    x3 = x.reshape(steps, 1, _E)

    out = pl.pallas_call(
        _kernel,
        out_shape=jax.ShapeDtypeStruct((steps, 1, _E), jnp.float32),
        grid=(steps,),
        in_specs=[
            pl.BlockSpec((1, 1, _E), lambda i: (i, 0, 0)),
            pl.BlockSpec((_BW, _NB), lambda i: (0, 0)),
            pl.BlockSpec((_VROWS, _NB), lambda i: (0, 0)),
        ],
        out_specs=pl.BlockSpec((1, 1, _E), lambda i: (i, 0, 0)),
        scratch_shapes=[
            pltpu.VMEM((_NB, _E), jnp.float32),
            pltpu.VMEM((_TROWS, _NB), jnp.bfloat16),
        ],
        compiler_params=pltpu.CompilerParams(
            dimension_semantics=("arbitrary",),
        ),
        name="stepnet_lookup",
    )(x3, bp_t, v_t)
    return out.reshape(B, 1)


# speedup vs baseline: 1.4555x; 1.1442x over previous
import jax
import jax.numpy as jnp
from jax.experimental import pallas as pl
from jax.experimental.pallas import tpu as pltpu

_NB = 64
_BW = 32
_E = 16384

_VROWS = _BW + 1
_V0 = 3 * _BW
_TROWS = _V0 + 2 * _BW + 2

_EXP_MASK = -65536


def _kernel(x_ref, bpt_ref, vt_ref, o_ref, bnd_s, tab_s):
    f32 = jnp.float32
    bf16 = jnp.bfloat16

    @pl.when(pl.program_id(0) == 0)
    def _prep():
        def split3(a):
            bits = jax.lax.bitcast_convert_type(a, jnp.int32)
            hi = jax.lax.bitcast_convert_type(bits & _EXP_MASK, f32)
            r1 = a - hi
            b1 = jax.lax.bitcast_convert_type(r1, jnp.int32)
            mid = jax.lax.bitcast_convert_type(b1 & _EXP_MASK, f32)
            lo = r1 - mid
            return hi, mid, lo

        def split2(a):
            bits = jax.lax.bitcast_convert_type(a, jnp.int32)
            hi = jax.lax.bitcast_convert_type(bits & _EXP_MASK, f32)
            return hi, a - hi

        bh, bm, bl = split3(bpt_ref[...])
        vt = vt_ref[...]
        dv = vt[1:_VROWS] - vt[: _BW]
        d1, d2 = split2(dv)
        v01, v02 = split2(vt[0:1])
        table_t = jnp.concatenate([bh, bm, bl, d1, d2, v01, v02], axis=0)
        tab_s[...] = table_t.astype(bf16)

        bnd_col = jnp.transpose(bpt_ref[_BW - 1 : _BW, :])
        bnd_s[...] = jnp.broadcast_to(bnd_col, (_NB, _E))

    xrow = x_ref[0]
    h1 = (xrow > bnd_s[...]).astype(bf16)
    h1p = jnp.concatenate([jnp.ones((1, _E), bf16), h1[: _NB - 1]], axis=0)
    onehot = h1p - h1

    g = jnp.dot(tab_s[...], onehot, preferred_element_type=f32)
    bp_row = (g[0:_BW] + g[_BW : 2 * _BW]) + g[2 * _BW : 3 * _BW]
    dv = g[_V0 : _V0 + _BW] + g[_V0 + _BW : _V0 + 2 * _BW]
    v0 = g[_V0 + 2 * _BW : _V0 + 2 * _BW + 1] + g[_V0 + 2 * _BW + 1 : _TROWS]

    cmp = (xrow > bp_row).astype(f32)
    sel = v0 + jnp.sum(cmp * dv, axis=0, keepdims=True)

    bp_last = bpt_ref[_BW - 1, _NB - 1]
    v_last = vt_ref[_BW, _NB - 1]
    out = sel + (xrow > bp_last).astype(f32) * v_last
    o_ref[...] = out.reshape(1, 1, _E)


def kernel(x, breakpoints, values):
    B = x.shape[0]
    n = breakpoints.shape[0]
    steps = B // _E

    bp_r = breakpoints.reshape(_NB, _BW)
    bp_t = bp_r.T
    v_main = values[:n, 0].reshape(_NB, _BW)
    v_ext = values[1 : n + 1, 0].reshape(_NB, _BW)
    v_t = jnp.concatenate([v_main, v_ext[:, _BW - 1 :]], axis=1).T

    x3 = x.reshape(steps, 1, _E)

    out = pl.pallas_call(
        _kernel,
        out_shape=jax.ShapeDtypeStruct((steps, 1, _E), jnp.float32),
        grid=(steps,),
        in_specs=[
            pl.BlockSpec((1, 1, _E), lambda i: (i, 0, 0)),
            pl.BlockSpec((_BW, _NB), lambda i: (0, 0)),
            pl.BlockSpec((_VROWS, _NB), lambda i: (0, 0)),
        ],
        out_specs=pl.BlockSpec((1, 1, _E), lambda i: (i, 0, 0)),
        scratch_shapes=[
            pltpu.VMEM((_NB, _E), jnp.float32),
            pltpu.VMEM((_TROWS, _NB), jnp.bfloat16),
        ],
        compiler_params=pltpu.CompilerParams(
            dimension_semantics=("arbitrary",),
        ),
        name="stepnet_lookup",
    )(x3, bp_t, v_t)
    return out.reshape(B, 1)
